# baseline (speedup 1.0000x reference)
import functools

import jax
import jax.numpy as jnp
from jax import lax
from jax.experimental import pallas as pl
from jax.experimental.pallas import tpu as pltpu
from jax.experimental.pallas import tpu_sc as plsc

B, C, H, W = 8, 768, 32, 32
HW = H * W
NPIX = B * HW
NREF = 4
LANES = 16
NC, NS = 2, 16
NW = NC * NS
NSPLIT = 2
BSPL = B // NSPLIT
NPIXS = BSPL * HW
GROUP = 8
GPW = NPIXS // NW // GROUP
NCHUNK = C // LANES
SCALE = 1.0 / float(C) ** 0.5


def _tc_body(x_ref, wq_ref, wk_ref, wv_ref, wo_ref, bq_ref, bk_ref, bv_ref,
             bo_ref, v_ref, s_ref, gidx_ref):
    b = pl.program_id(0)
    xb = x_ref[...]
    q = jnp.dot(xb, wq_ref[...], preferred_element_type=jnp.float32) + bq_ref[...]
    k = jnp.dot(xb, wk_ref[...], preferred_element_type=jnp.float32) + bk_ref[...]
    v_ref[...] = jnp.dot(xb, wv_ref[...], preferred_element_type=jnp.float32) + bv_ref[...]
    s_ref[...] = lax.dot_general(q, k, (((1,), (1,)), ((), ())),
                                 preferred_element_type=jnp.float32)
    off = jnp.dot(q, wo_ref[...], preferred_element_type=jnp.float32) + bo_ref[...]
    p = lax.broadcasted_iota(jnp.int32, (HW, 1), 0)
    ypix = (p // W).astype(jnp.float32)
    xpix = (p % W).astype(jnp.float32)
    cols = []
    for r in range(NREF):
        rx = jnp.floor(jnp.clip(xpix + off[:, 2 * r:2 * r + 1], 0.0, W - 1.0))
        ry = jnp.floor(jnp.clip(ypix + off[:, 2 * r + 1:2 * r + 2], 0.0, H - 1.0))
        cols.append(b * HW + ry.astype(jnp.int32) * W + rx.astype(jnp.int32))
    gidx_ref[...] = jnp.concatenate(cols, axis=1)


_tc_call = pl.pallas_call(
    _tc_body,
    grid=(BSPL,),
    in_specs=[
        pl.BlockSpec((HW, C), lambda i: (i, 0)),
        pl.BlockSpec((C, C), lambda i: (0, 0)),
        pl.BlockSpec((C, C), lambda i: (0, 0)),
        pl.BlockSpec((C, C), lambda i: (0, 0)),
        pl.BlockSpec((C, 2 * NREF), lambda i: (0, 0)),
        pl.BlockSpec((1, C), lambda i: (0, 0)),
        pl.BlockSpec((1, C), lambda i: (0, 0)),
        pl.BlockSpec((1, C), lambda i: (0, 0)),
        pl.BlockSpec((1, 2 * NREF), lambda i: (0, 0)),
    ],
    out_specs=[
        pl.BlockSpec((HW, C), lambda i: (i, 0)),
        pl.BlockSpec((HW, HW), lambda i: (i, 0)),
        pl.BlockSpec((HW, NREF), lambda i: (i, 0)),
    ],
    out_shape=[
        jax.ShapeDtypeStruct((NPIXS, C), jnp.float32),
        jax.ShapeDtypeStruct((NPIXS, HW), jnp.float32),
        jax.ShapeDtypeStruct((NPIXS, NREF), jnp.int32),
    ],
)


def _lane_splat(vec, lane):
    perm = jnp.broadcast_to(lane, (LANES,))
    return lax.gather(
        vec, perm[:, None],
        lax.GatherDimensionNumbers(offset_dims=(), collapsed_slice_dims=(0,),
                                   start_index_map=(0,)),
        slice_sizes=(1,), mode=lax.GatherScatterMode.PROMISE_IN_BOUNDS)


def _sc_body(v2, s2, gidxf, out2, idx_v, sidx_v, vrows, s_v, out_v,
             sem_in, sem_out):
    wid = lax.axis_index("s") * NC + lax.axis_index("c")
    g0 = wid * GPW

    def issue(grp, b):
        base = grp * GROUP
        pltpu.sync_copy(gidxf.at[pl.ds(grp * (GROUP * NREF), GROUP * NREF)],
                        idx_v.at[b])
        pltpu.async_copy(v2.at[idx_v.at[b]], vrows.at[b], sem_in)
        sidx_v[b, pl.ds(0, LANES)] = (jnp.broadcast_to(base, (LANES,))
                                      + lax.iota(jnp.int32, LANES))
        pltpu.async_copy(s2.at[sidx_v.at[b, pl.ds(0, GROUP)]], s_v.at[b],
                         sem_in)

    def wait_in(b):
        pltpu.make_async_copy(v2.at[idx_v.at[b]], vrows.at[b], sem_in).wait()
        pltpu.make_async_copy(s2.at[sidx_v.at[b, pl.ds(0, GROUP)]],
                              s_v.at[b], sem_in).wait()

    def drain_out(b):
        pltpu.make_async_copy(out_v.at[b], out2.at[pl.ds(0, GROUP)],
                              sem_out).wait()

    issue(g0, 0)

    def pair(gp, _):
        for b in range(2):
            g = gp * 2 + b
            grp = g0 + g
            base = grp * GROUP
            wait_in(b)

            @pl.when(g + 1 < GPW)
            def _():
                issue(grp + 1, 1 - b)

            @pl.when(g >= 2)
            def _():
                drain_out(b)

            chunks = [idx_v[b, pl.ds(c * LANES, LANES)] for c in range(2)]
            for p in range(GROUP):
                avs = []
                for r in range(NREF):
                    j = p * NREF + r
                    li = chunks[j // LANES][j % LANES] & (HW - 1)
                    start = pl.multiple_of(li & ~(LANES - 1), LANES)
                    cvec = s_v[b, p, pl.ds(start, LANES)]
                    zv = _lane_splat(cvec, li & (LANES - 1)) * SCALE
                    avs.append(1.0 / (1.0 + jnp.exp(-zv)))
                j0 = p * NREF

                def wchunk(c8, _, b=b, p=p, j0=j0, avs=avs):
                    for u in range(8):
                        sl = pl.ds(pl.multiple_of(c8 * (8 * LANES) + u * LANES,
                                                  LANES), LANES)
                        o = avs[0] * vrows[b, j0, sl]
                        for r in range(1, NREF):
                            o = o + avs[r] * vrows[b, j0 + r, sl]
                        out_v[b, p, sl] = o
                    return 0

                lax.fori_loop(0, NCHUNK // 8, wchunk, 0)
            pltpu.async_copy(out_v.at[b], out2.at[pl.ds(base, GROUP)], sem_out)
        return 0

    lax.fori_loop(0, GPW // 2, pair, 0)
    drain_out(0)
    drain_out(1)


@functools.cache
def _sc_call():
    return pl.kernel(
        _sc_body,
        out_type=jax.ShapeDtypeStruct((NPIXS, C), jnp.float32),
        mesh=plsc.VectorSubcoreMesh(core_axis_name="c", subcore_axis_name="s"),
        scratch_types=[
            pltpu.VMEM((2, GROUP * NREF), jnp.int32),
            pltpu.VMEM((2, LANES), jnp.int32),
            pltpu.VMEM((2, GROUP * NREF, C), jnp.float32),
            pltpu.VMEM((2, GROUP, HW), jnp.float32),
            pltpu.VMEM((2, GROUP, C), jnp.float32),
            pltpu.SemaphoreType.DMA,
            pltpu.SemaphoreType.DMA,
        ],
    )


def kernel(x, Wq, bq, Wk, bk, Wv, bv, Wo, bo):
    x2 = x.reshape(B, C, HW).transpose(0, 2, 1).reshape(NPIX, C)
    outs = []
    for sp in range(NSPLIT):
        xs = x2[sp * NPIXS:(sp + 1) * NPIXS]
        v2, s2, gidx = _tc_call(xs, Wq.T, Wk.T, Wv.T, Wo.T, bq[None, :],
                                bk[None, :], bv[None, :], bo[None, :])
        outs.append(_sc_call()(v2, s2, gidx.reshape(NPIXS * NREF)))
    out2 = jnp.concatenate(outs, axis=0)
    return out2.reshape(B, HW, C).transpose(0, 2, 1).reshape(B, C, H, W)


# speedup vs baseline: 1.0026x; 1.0026x over previous
import functools

import jax
import jax.numpy as jnp
from jax import lax
from jax.experimental import pallas as pl
from jax.experimental.pallas import tpu as pltpu
from jax.experimental.pallas import tpu_sc as plsc

B, C, H, W = 8, 768, 32, 32
HW = H * W
NPIX = B * HW
NREF = 4
LANES = 16
NC, NS = 2, 16
NW = NC * NS
NSPLIT = 2
BSPL = B // NSPLIT
NPIXS = BSPL * HW
GROUP = 8
GPW = NPIXS // NW // GROUP
NCHUNK = C // LANES
SCALE = 1.0 / float(C) ** 0.5


def _tc_body(x_ref, wq_ref, wk_ref, wv_ref, wo_ref, bq_ref, bk_ref, bv_ref,
             bo_ref, v_ref, s_ref, gidx_ref):
    b = pl.program_id(0)
    xb = x_ref[...]
    q = jnp.dot(xb, wq_ref[...], preferred_element_type=jnp.float32) + bq_ref[...]
    k = jnp.dot(xb, wk_ref[...], preferred_element_type=jnp.float32) + bk_ref[...]
    v_ref[...] = jnp.dot(xb, wv_ref[...], preferred_element_type=jnp.float32) + bv_ref[...]
    s_ref[...] = lax.dot_general(q, k, (((1,), (1,)), ((), ())),
                                 preferred_element_type=jnp.float32)
    off = jnp.dot(q, wo_ref[...], preferred_element_type=jnp.float32) + bo_ref[...]
    p = lax.broadcasted_iota(jnp.int32, (HW, 1), 0)
    ypix = (p // W).astype(jnp.float32)
    xpix = (p % W).astype(jnp.float32)
    cols = []
    for r in range(NREF):
        rx = jnp.floor(jnp.clip(xpix + off[:, 2 * r:2 * r + 1], 0.0, W - 1.0))
        ry = jnp.floor(jnp.clip(ypix + off[:, 2 * r + 1:2 * r + 2], 0.0, H - 1.0))
        cols.append(b * HW + ry.astype(jnp.int32) * W + rx.astype(jnp.int32))
    gidx_ref[...] = jnp.concatenate(cols, axis=1)


_tc_call = pl.pallas_call(
    _tc_body,
    grid=(BSPL,),
    in_specs=[
        pl.BlockSpec((HW, C), lambda i: (i, 0)),
        pl.BlockSpec((C, C), lambda i: (0, 0)),
        pl.BlockSpec((C, C), lambda i: (0, 0)),
        pl.BlockSpec((C, C), lambda i: (0, 0)),
        pl.BlockSpec((C, 2 * NREF), lambda i: (0, 0)),
        pl.BlockSpec((1, C), lambda i: (0, 0)),
        pl.BlockSpec((1, C), lambda i: (0, 0)),
        pl.BlockSpec((1, C), lambda i: (0, 0)),
        pl.BlockSpec((1, 2 * NREF), lambda i: (0, 0)),
    ],
    out_specs=[
        pl.BlockSpec((HW, C), lambda i: (i, 0)),
        pl.BlockSpec((HW, HW), lambda i: (i, 0)),
        pl.BlockSpec((HW, NREF), lambda i: (i, 0)),
    ],
    out_shape=[
        jax.ShapeDtypeStruct((NPIXS, C), jnp.float32),
        jax.ShapeDtypeStruct((NPIXS, HW), jnp.float32),
        jax.ShapeDtypeStruct((NPIXS, NREF), jnp.int32),
    ],
)


def _lane_splat(vec, lane):
    perm = jnp.broadcast_to(lane, (LANES,))
    return lax.gather(
        vec, perm[:, None],
        lax.GatherDimensionNumbers(offset_dims=(), collapsed_slice_dims=(0,),
                                   start_index_map=(0,)),
        slice_sizes=(1,), mode=lax.GatherScatterMode.PROMISE_IN_BOUNDS)


def _sc_body(v2, s2, gidxf, out2, idx_v, sidx_v, vrows, s_v, out_v,
             sem_in, sem_out):
    wid = lax.axis_index("s") * NC + lax.axis_index("c")
    g0 = wid * GPW

    def issue(grp, b):
        base = grp * GROUP
        pltpu.sync_copy(gidxf.at[pl.ds(grp * (GROUP * NREF), GROUP * NREF)],
                        idx_v.at[b])
        pltpu.async_copy(v2.at[idx_v.at[b]], vrows.at[b], sem_in)
        sidx_v[b, pl.ds(0, LANES)] = (jnp.broadcast_to(base, (LANES,))
                                      + lax.iota(jnp.int32, LANES))
        pltpu.async_copy(s2.at[sidx_v.at[b, pl.ds(0, GROUP)]], s_v.at[b],
                         sem_in)

    def wait_in(b):
        pltpu.make_async_copy(v2.at[idx_v.at[b]], vrows.at[b], sem_in).wait()
        pltpu.make_async_copy(s2.at[sidx_v.at[b, pl.ds(0, GROUP)]],
                              s_v.at[b], sem_in).wait()

    def drain_out(b):
        pltpu.make_async_copy(out_v.at[b], out2.at[pl.ds(0, GROUP)],
                              sem_out).wait()

    issue(g0, 0)

    def pair(gp, _):
        for b in range(2):
            g = gp * 2 + b
            grp = g0 + g
            base = grp * GROUP
            wait_in(b)

            @pl.when(g + 1 < GPW)
            def _():
                issue(grp + 1, 1 - b)

            @pl.when(g >= 2)
            def _():
                drain_out(b)

            chunks = [idx_v[b, pl.ds(c * LANES, LANES)] for c in range(2)]
            for p in range(GROUP):
                avs = []
                for r in range(NREF):
                    j = p * NREF + r
                    li = chunks[j // LANES][j % LANES] & (HW - 1)
                    start = pl.multiple_of(li & ~(LANES - 1), LANES)
                    cvec = s_v[b, p, pl.ds(start, LANES)]
                    zv = _lane_splat(cvec, li & (LANES - 1)) * SCALE
                    avs.append(1.0 / (1.0 + jnp.exp(-zv)))
                j0 = p * NREF

                def wchunk(c8, _, b=b, p=p, j0=j0, avs=avs):
                    for u in range(8):
                        sl = pl.ds(pl.multiple_of(c8 * (8 * LANES) + u * LANES,
                                                  LANES), LANES)
                        o = avs[0] * vrows[b, j0, sl]
                        for r in range(1, NREF):
                            o = o + avs[r] * vrows[b, j0 + r, sl]
                        out_v[b, p, sl] = o
                    return 0

                lax.fori_loop(0, NCHUNK // 8, wchunk, 0)
            pltpu.async_copy(out_v.at[b], out2.at[pl.ds(base, GROUP)], sem_out)
        return 0

    lax.fori_loop(0, GPW // 2, pair, 0)
    drain_out(0)
    drain_out(1)


@functools.cache
def _sc_call():
    return pl.kernel(
        _sc_body,
        out_type=jax.ShapeDtypeStruct((NPIXS, C), jnp.float32),
        mesh=plsc.VectorSubcoreMesh(core_axis_name="c", subcore_axis_name="s"),
        scratch_types=[
            pltpu.VMEM((2, GROUP * NREF), jnp.int32),
            pltpu.VMEM((2, LANES), jnp.int32),
            pltpu.VMEM((2, GROUP * NREF, C), jnp.float32),
            pltpu.VMEM((2, GROUP, HW), jnp.float32),
            pltpu.VMEM((2, GROUP, C), jnp.float32),
            pltpu.SemaphoreType.DMA,
            pltpu.SemaphoreType.DMA,
        ],
    )


def kernel(x, Wq, bq, Wk, bk, Wv, bv, Wo, bo):
    x2 = x.reshape(B, C, HW).transpose(0, 2, 1).reshape(NPIX, C)
    tc_outs = []
    for sp in range(NSPLIT):
        xs = x2[sp * NPIXS:(sp + 1) * NPIXS]
        tc_outs.append(_tc_call(xs, Wq.T, Wk.T, Wv.T, Wo.T, bq[None, :],
                                bk[None, :], bv[None, :], bo[None, :]))
    outs = [_sc_call()(v2, s2, gidx.reshape(NPIXS * NREF))
            for v2, s2, gidx in tc_outs]
    out2 = jnp.concatenate(outs, axis=0)
    return out2.reshape(B, HW, C).transpose(0, 2, 1).reshape(B, C, H, W)


# speedup vs baseline: 1.1302x; 1.1273x over previous
import functools

import jax
import jax.numpy as jnp
from jax import lax
from jax.experimental import pallas as pl
from jax.experimental.pallas import tpu as pltpu
from jax.experimental.pallas import tpu_sc as plsc

B, C, H, W = 8, 768, 32, 32
HW = H * W
NPIX = B * HW
NREF = 4
LANES = 16
NC, NS = 2, 16
NW = NC * NS
NSPLIT = 1
BSPL = B // NSPLIT
NPIXS = BSPL * HW
GROUP = 8
GPW = NPIXS // NW // GROUP
NCHUNK = C // LANES
SCALE = 1.0 / float(C) ** 0.5


def _tc_body(x_ref, wq_ref, wk_ref, wv_ref, wo_ref, bq_ref, bk_ref, bv_ref,
             bo_ref, v_ref, s_ref, gidx_ref):
    b = pl.program_id(0)
    xb = x_ref[...]
    q = jnp.dot(xb, wq_ref[...], preferred_element_type=jnp.float32) + bq_ref[...]
    xb16 = xb.astype(jnp.bfloat16)
    k = jnp.dot(xb16, wk_ref[...].astype(jnp.bfloat16),
                preferred_element_type=jnp.float32) + bk_ref[...]
    v_ref[...] = jnp.dot(xb16, wv_ref[...].astype(jnp.bfloat16),
                         preferred_element_type=jnp.float32) + bv_ref[...]
    s_ref[...] = lax.dot_general(q.astype(jnp.bfloat16), k.astype(jnp.bfloat16),
                                 (((1,), (1,)), ((), ())),
                                 preferred_element_type=jnp.float32)
    off = jnp.dot(q, wo_ref[...], preferred_element_type=jnp.float32) + bo_ref[...]
    p = lax.broadcasted_iota(jnp.int32, (HW, 1), 0)
    ypix = (p // W).astype(jnp.float32)
    xpix = (p % W).astype(jnp.float32)
    cols = []
    for r in range(NREF):
        rx = jnp.floor(jnp.clip(xpix + off[:, 2 * r:2 * r + 1], 0.0, W - 1.0))
        ry = jnp.floor(jnp.clip(ypix + off[:, 2 * r + 1:2 * r + 2], 0.0, H - 1.0))
        cols.append(b * HW + ry.astype(jnp.int32) * W + rx.astype(jnp.int32))
    gidx_ref[...] = jnp.concatenate(cols, axis=1)


_tc_call = pl.pallas_call(
    _tc_body,
    grid=(BSPL,),
    in_specs=[
        pl.BlockSpec((HW, C), lambda i: (i, 0)),
        pl.BlockSpec((C, C), lambda i: (0, 0)),
        pl.BlockSpec((C, C), lambda i: (0, 0)),
        pl.BlockSpec((C, C), lambda i: (0, 0)),
        pl.BlockSpec((C, 2 * NREF), lambda i: (0, 0)),
        pl.BlockSpec((1, C), lambda i: (0, 0)),
        pl.BlockSpec((1, C), lambda i: (0, 0)),
        pl.BlockSpec((1, C), lambda i: (0, 0)),
        pl.BlockSpec((1, 2 * NREF), lambda i: (0, 0)),
    ],
    out_specs=[
        pl.BlockSpec((HW, C), lambda i: (i, 0)),
        pl.BlockSpec((HW, HW), lambda i: (i, 0)),
        pl.BlockSpec((HW, NREF), lambda i: (i, 0)),
    ],
    out_shape=[
        jax.ShapeDtypeStruct((NPIXS, C), jnp.float32),
        jax.ShapeDtypeStruct((NPIXS, HW), jnp.float32),
        jax.ShapeDtypeStruct((NPIXS, NREF), jnp.int32),
    ],
)


def _lane_splat(vec, lane):
    perm = jnp.broadcast_to(lane, (LANES,))
    return lax.gather(
        vec, perm[:, None],
        lax.GatherDimensionNumbers(offset_dims=(), collapsed_slice_dims=(0,),
                                   start_index_map=(0,)),
        slice_sizes=(1,), mode=lax.GatherScatterMode.PROMISE_IN_BOUNDS)


def _sc_body(v2, s2, gidxf, out2, idx_v, sidx_v, vrows, s_v, out_v,
             sem_in, sem_out):
    wid = lax.axis_index("s") * NC + lax.axis_index("c")
    g0 = wid * GPW

    def issue(grp, b):
        base = grp * GROUP
        pltpu.sync_copy(gidxf.at[pl.ds(grp * (GROUP * NREF), GROUP * NREF)],
                        idx_v.at[b])
        pltpu.async_copy(v2.at[idx_v.at[b]], vrows.at[b], sem_in)
        sidx_v[b, pl.ds(0, LANES)] = (jnp.broadcast_to(base, (LANES,))
                                      + lax.iota(jnp.int32, LANES))
        pltpu.async_copy(s2.at[sidx_v.at[b, pl.ds(0, GROUP)]], s_v.at[b],
                         sem_in)

    def wait_in(b):
        pltpu.make_async_copy(v2.at[idx_v.at[b]], vrows.at[b], sem_in).wait()
        pltpu.make_async_copy(s2.at[sidx_v.at[b, pl.ds(0, GROUP)]],
                              s_v.at[b], sem_in).wait()

    def drain_out(b):
        pltpu.make_async_copy(out_v.at[b], out2.at[pl.ds(0, GROUP)],
                              sem_out).wait()

    issue(g0, 0)

    def pair(gp, _):
        for b in range(2):
            g = gp * 2 + b
            grp = g0 + g
            base = grp * GROUP
            wait_in(b)

            @pl.when(g + 1 < GPW)
            def _():
                issue(grp + 1, 1 - b)

            @pl.when(g >= 2)
            def _():
                drain_out(b)

            chunks = [idx_v[b, pl.ds(c * LANES, LANES)] for c in range(2)]
            for p in range(GROUP):
                avs = []
                for r in range(NREF):
                    j = p * NREF + r
                    li = chunks[j // LANES][j % LANES] & (HW - 1)
                    start = pl.multiple_of(li & ~(LANES - 1), LANES)
                    cvec = s_v[b, p, pl.ds(start, LANES)]
                    zv = _lane_splat(cvec, li & (LANES - 1)) * SCALE
                    avs.append(1.0 / (1.0 + jnp.exp(-zv)))
                j0 = p * NREF

                def wchunk(c8, _, b=b, p=p, j0=j0, avs=avs):
                    for u in range(8):
                        sl = pl.ds(pl.multiple_of(c8 * (8 * LANES) + u * LANES,
                                                  LANES), LANES)
                        o = avs[0] * vrows[b, j0, sl]
                        for r in range(1, NREF):
                            o = o + avs[r] * vrows[b, j0 + r, sl]
                        out_v[b, p, sl] = o
                    return 0

                lax.fori_loop(0, NCHUNK // 8, wchunk, 0)
            pltpu.async_copy(out_v.at[b], out2.at[pl.ds(base, GROUP)], sem_out)
        return 0

    lax.fori_loop(0, GPW // 2, pair, 0)
    drain_out(0)
    drain_out(1)


@functools.cache
def _sc_call():
    return pl.kernel(
        _sc_body,
        out_type=jax.ShapeDtypeStruct((NPIXS, C), jnp.float32),
        mesh=plsc.VectorSubcoreMesh(core_axis_name="c", subcore_axis_name="s"),
        scratch_types=[
            pltpu.VMEM((2, GROUP * NREF), jnp.int32),
            pltpu.VMEM((2, LANES), jnp.int32),
            pltpu.VMEM((2, GROUP * NREF, C), jnp.float32),
            pltpu.VMEM((2, GROUP, HW), jnp.float32),
            pltpu.VMEM((2, GROUP, C), jnp.float32),
            pltpu.SemaphoreType.DMA,
            pltpu.SemaphoreType.DMA,
        ],
    )


def kernel(x, Wq, bq, Wk, bk, Wv, bv, Wo, bo):
    x2 = x.reshape(B, C, HW).transpose(0, 2, 1).reshape(NPIX, C)
    tc_outs = []
    for sp in range(NSPLIT):
        xs = x2[sp * NPIXS:(sp + 1) * NPIXS]
        tc_outs.append(_tc_call(xs, Wq.T, Wk.T, Wv.T, Wo.T, bq[None, :],
                                bk[None, :], bv[None, :], bo[None, :]))
    outs = [_sc_call()(v2, s2, gidx.reshape(NPIXS * NREF))
            for v2, s2, gidx in tc_outs]
    out2 = jnp.concatenate(outs, axis=0)
    return out2.reshape(B, HW, C).transpose(0, 2, 1).reshape(B, C, H, W)


# speedup vs baseline: 1.1308x; 1.0005x over previous
import functools

import jax
import jax.numpy as jnp
from jax import lax
from jax.experimental import pallas as pl
from jax.experimental.pallas import tpu as pltpu
from jax.experimental.pallas import tpu_sc as plsc

B, C, H, W = 8, 768, 32, 32
HW = H * W
NPIX = B * HW
NREF = 4
LANES = 16
NC, NS = 2, 16
NW = NC * NS
PPW = NPIX // NW
GROUP = 8
GPW = PPW // GROUP
NCHUNK = C // LANES
SCALE = 1.0 / float(C) ** 0.5


def _tc_body(x_ref, wq_ref, wk_ref, wv_ref, wo_ref, bq_ref, bk_ref, bv_ref,
             bo_ref, v_ref, s_ref, gidx_ref):
    b = pl.program_id(0)
    xt = lax.transpose(x_ref[0], (1, 0))
    xb16 = xt.astype(jnp.bfloat16)
    q = jnp.dot(xt, wq_ref[...], preferred_element_type=jnp.float32) + bq_ref[...]
    k = jnp.dot(xb16, wk_ref[...].astype(jnp.bfloat16),
                preferred_element_type=jnp.float32) + bk_ref[...]
    v_ref[...] = jnp.dot(xb16, wv_ref[...].astype(jnp.bfloat16),
                         preferred_element_type=jnp.float32) + bv_ref[...]
    s_ref[...] = lax.dot_general(q.astype(jnp.bfloat16), k.astype(jnp.bfloat16),
                                 (((1,), (1,)), ((), ())),
                                 preferred_element_type=jnp.float32)
    off = lax.dot_general(wo_ref[...], q, (((1,), (1,)), ((), ())),
                          preferred_element_type=jnp.float32) + bo_ref[...]
    p = lax.broadcasted_iota(jnp.int32, (1, HW), 1)
    ypix = (p // W).astype(jnp.float32)
    xpix = (p % W).astype(jnp.float32)
    for r in range(NREF):
        rx = jnp.floor(jnp.clip(xpix + off[2 * r:2 * r + 1, :], 0.0, W - 1.0))
        ry = jnp.floor(jnp.clip(ypix + off[2 * r + 1:2 * r + 2, :], 0.0, H - 1.0))
        gidx_ref[r:r + 1, :] = (b * HW + ry.astype(jnp.int32) * W
                                + rx.astype(jnp.int32))


_tc_call = pl.pallas_call(
    _tc_body,
    grid=(B,),
    in_specs=[
        pl.BlockSpec((1, C, HW), lambda i: (i, 0, 0)),
        pl.BlockSpec((C, C), lambda i: (0, 0)),
        pl.BlockSpec((C, C), lambda i: (0, 0)),
        pl.BlockSpec((C, C), lambda i: (0, 0)),
        pl.BlockSpec((2 * NREF, C), lambda i: (0, 0)),
        pl.BlockSpec((1, C), lambda i: (0, 0)),
        pl.BlockSpec((1, C), lambda i: (0, 0)),
        pl.BlockSpec((1, C), lambda i: (0, 0)),
        pl.BlockSpec((2 * NREF, 1), lambda i: (0, 0)),
    ],
    out_specs=[
        pl.BlockSpec((HW, C), lambda i: (i, 0)),
        pl.BlockSpec((HW, HW), lambda i: (i, 0)),
        pl.BlockSpec((NREF, HW), lambda i: (0, i)),
    ],
    out_shape=[
        jax.ShapeDtypeStruct((NPIX, C), jnp.float32),
        jax.ShapeDtypeStruct((NPIX, HW), jnp.float32),
        jax.ShapeDtypeStruct((NREF, NPIX), jnp.int32),
    ],
)


def _lane_perm(vec, perm):
    return lax.gather(
        vec, perm[:, None],
        lax.GatherDimensionNumbers(offset_dims=(), collapsed_slice_dims=(0,),
                                   start_index_map=(0,)),
        slice_sizes=(1,), mode=lax.GatherScatterMode.PROMISE_IN_BOUNDS)


def _lane_splat(vec, lane):
    return _lane_perm(vec, jnp.broadcast_to(lane, (LANES,)))


def _sc_body(v2, s2, gidxt, out2, idx_all, vidx_v, sidx_v, vrows, s_v, out_v,
             sem_in, sem_out):
    wid = lax.axis_index("s") * NC + lax.axis_index("c")
    g0 = wid * GPW
    pltpu.sync_copy(gidxt.at[:, pl.ds(wid * PPW, PPW)], idx_all)

    def issue(grp, g, b):
        base = grp * GROUP
        coff = pl.multiple_of((g >> 1) * LANES, LANES)
        half = (g & 1) * GROUP
        perm = (lax.iota(jnp.int32, LANES) & (GROUP - 1)) + half
        cv = [_lane_perm(idx_all[r, pl.ds(coff, LANES)], perm)
              for r in range(NREF)]
        lo = lax.iota(jnp.int32, LANES) < GROUP
        vidx_v[b, pl.ds(0, LANES)] = jnp.where(lo, cv[0], cv[1])
        vidx_v[b, pl.ds(LANES, LANES)] = jnp.where(lo, cv[2], cv[3])
        pltpu.async_copy(v2.at[vidx_v.at[b]], vrows.at[b], sem_in)
        sidx_v[b, pl.ds(0, LANES)] = (jnp.broadcast_to(base, (LANES,))
                                      + lax.iota(jnp.int32, LANES))
        pltpu.async_copy(s2.at[sidx_v.at[b, pl.ds(0, GROUP)]], s_v.at[b],
                         sem_in)

    def wait_in(b):
        pltpu.make_async_copy(v2.at[vidx_v.at[b]], vrows.at[b], sem_in).wait()
        pltpu.make_async_copy(s2.at[sidx_v.at[b, pl.ds(0, GROUP)]],
                              s_v.at[b], sem_in).wait()

    def drain_out(b):
        pltpu.make_async_copy(out_v.at[b], out2.at[pl.ds(0, GROUP)],
                              sem_out).wait()

    issue(g0, 0, 0)

    def pair(gp, _):
        for b in range(2):
            g = gp * 2 + b
            grp = g0 + g
            base = grp * GROUP
            wait_in(b)

            @pl.when(g + 1 < GPW)
            def _():
                issue(grp + 1, g + 1, 1 - b)

            @pl.when(g >= 2)
            def _():
                drain_out(b)

            chunks = [vidx_v[b, pl.ds(c * LANES, LANES)] for c in range(2)]
            for p in range(GROUP):
                avs = []
                for r in range(NREF):
                    li = chunks[r // 2][(r & 1) * GROUP + p] & (HW - 1)
                    start = pl.multiple_of(li & ~(LANES - 1), LANES)
                    cvec = s_v[b, p, pl.ds(start, LANES)]
                    zv = _lane_splat(cvec, li & (LANES - 1)) * SCALE
                    avs.append(1.0 / (1.0 + jnp.exp(-zv)))

                def wchunk(c8, _, b=b, p=p, avs=avs):
                    for u in range(8):
                        sl = pl.ds(pl.multiple_of(c8 * (8 * LANES) + u * LANES,
                                                  LANES), LANES)
                        o = avs[0] * vrows[b, p, sl]
                        for r in range(1, NREF):
                            o = o + avs[r] * vrows[b, r * GROUP + p, sl]
                        out_v[b, p, sl] = o
                    return 0

                lax.fori_loop(0, NCHUNK // 8, wchunk, 0)
            pltpu.async_copy(out_v.at[b], out2.at[pl.ds(base, GROUP)], sem_out)
        return 0

    lax.fori_loop(0, GPW // 2, pair, 0)
    drain_out(0)
    drain_out(1)


@functools.cache
def _sc_call():
    return pl.kernel(
        _sc_body,
        out_type=jax.ShapeDtypeStruct((NPIX, C), jnp.float32),
        mesh=plsc.VectorSubcoreMesh(core_axis_name="c", subcore_axis_name="s"),
        scratch_types=[
            pltpu.VMEM((NREF, PPW), jnp.int32),
            pltpu.VMEM((2, GROUP * NREF), jnp.int32),
            pltpu.VMEM((2, LANES), jnp.int32),
            pltpu.VMEM((2, GROUP * NREF, C), jnp.float32),
            pltpu.VMEM((2, GROUP, HW), jnp.float32),
            pltpu.VMEM((2, GROUP, C), jnp.float32),
            pltpu.SemaphoreType.DMA,
            pltpu.SemaphoreType.DMA,
        ],
    )


def kernel(x, Wq, bq, Wk, bk, Wv, bv, Wo, bo):
    x3 = x.reshape(B, C, HW)
    v2, s2, gidxt = _tc_call(x3, Wq.T, Wk.T, Wv.T, Wo, bq[None, :],
                             bk[None, :], bv[None, :], bo[:, None])
    out2 = _sc_call()(v2, s2, gidxt)
    return out2.reshape(B, HW, C).transpose(0, 2, 1).reshape(B, C, H, W)


# speedup vs baseline: 1.2560x; 1.1108x over previous
import functools

import jax
import jax.numpy as jnp
from jax import lax
from jax.experimental import pallas as pl
from jax.experimental.pallas import tpu as pltpu
from jax.experimental.pallas import tpu_sc as plsc

B, C, H, W = 8, 768, 32, 32
HW = H * W
NPIX = B * HW
NREF = 4
LANES = 16
NC, NS = 2, 16
NW = NC * NS
PPW = NPIX // NW
GROUP = 8
GPW = PPW // GROUP
SCALE = 1.0 / float(C) ** 0.5
NT = (((1,), (1,)), ((), ()))


def _tc1_body(x_ref, wq_ref, wk_ref, wo_ref, bq_ref, bk_ref, bo_ref,
              s_ref, gidx_ref):
    b = pl.program_id(0)
    xt = lax.transpose(x_ref[0], (1, 0))
    q = lax.dot_general(xt, wq_ref[...], NT,
                        preferred_element_type=jnp.float32) + bq_ref[...]
    k = lax.dot_general(xt.astype(jnp.bfloat16),
                        wk_ref[...].astype(jnp.bfloat16), NT,
                        preferred_element_type=jnp.float32) + bk_ref[...]
    s_ref[...] = lax.dot_general(q.astype(jnp.bfloat16), k.astype(jnp.bfloat16),
                                 NT, preferred_element_type=jnp.float32)
    off = lax.dot_general(wo_ref[...], q, NT,
                          preferred_element_type=jnp.float32) + bo_ref[...]
    p = lax.broadcasted_iota(jnp.int32, (1, HW), 1)
    ypix = (p // W).astype(jnp.float32)
    xpix = (p % W).astype(jnp.float32)
    for r in range(NREF):
        rx = jnp.floor(jnp.clip(xpix + off[2 * r:2 * r + 1, :], 0.0, W - 1.0))
        ry = jnp.floor(jnp.clip(ypix + off[2 * r + 1:2 * r + 2, :], 0.0, H - 1.0))
        gidx_ref[r:r + 1, :] = (b * HW + ry.astype(jnp.int32) * W
                                + rx.astype(jnp.int32))


_tc1_call = pl.pallas_call(
    _tc1_body,
    grid=(B,),
    in_specs=[
        pl.BlockSpec((1, C, HW), lambda i: (i, 0, 0)),
        pl.BlockSpec((C, C), lambda i: (0, 0)),
        pl.BlockSpec((C, C), lambda i: (0, 0)),
        pl.BlockSpec((2 * NREF, C), lambda i: (0, 0)),
        pl.BlockSpec((1, C), lambda i: (0, 0)),
        pl.BlockSpec((1, C), lambda i: (0, 0)),
        pl.BlockSpec((2 * NREF, 1), lambda i: (0, 0)),
    ],
    out_specs=[
        pl.BlockSpec((HW, HW), lambda i: (i, 0)),
        pl.BlockSpec((NREF, HW), lambda i: (0, i)),
    ],
    out_shape=[
        jax.ShapeDtypeStruct((NPIX, HW), jnp.float32),
        jax.ShapeDtypeStruct((NREF, NPIX), jnp.int32),
    ],
)


def _lane_perm(vec, perm):
    return lax.gather(
        vec, perm[:, None],
        lax.GatherDimensionNumbers(offset_dims=(), collapsed_slice_dims=(0,),
                                   start_index_map=(0,)),
        slice_sizes=(1,), mode=lax.GatherScatterMode.PROMISE_IN_BOUNDS)


def _lane_splat(vec, lane):
    return _lane_perm(vec, jnp.broadcast_to(lane, (LANES,)))


def _sc_body(s2, gidxt, attz, idx_all, sidx_v, s_v, att_all, sem_in):
    wid = lax.axis_index("s") * NC + lax.axis_index("c")
    g0 = wid * GPW
    pltpu.sync_copy(gidxt.at[:, pl.ds(wid * PPW, PPW)], idx_all)

    def issue(grp, b):
        base = grp * GROUP
        sidx_v[b, pl.ds(0, LANES)] = (jnp.broadcast_to(base, (LANES,))
                                      + lax.iota(jnp.int32, LANES))
        pltpu.async_copy(s2.at[sidx_v.at[b, pl.ds(0, GROUP)]], s_v.at[b],
                         sem_in)

    def wait_in(b):
        pltpu.make_async_copy(s2.at[sidx_v.at[b, pl.ds(0, GROUP)]],
                              s_v.at[b], sem_in).wait()

    issue(g0, 0)
    lane_iota = lax.iota(jnp.int32, LANES)

    def pair(gp, _):
        acc = [jnp.zeros((LANES,), jnp.float32) for _ in range(NREF)]
        coff = pl.multiple_of(gp * LANES, LANES)
        cv = [idx_all[r, pl.ds(coff, LANES)] for r in range(NREF)]
        for b in range(2):
            g = gp * 2 + b
            grp = g0 + g
            wait_in(b)

            @pl.when(g + 1 < GPW)
            def _():
                issue(grp + 1, 1 - b)

            for p in range(GROUP):
                lane = b * GROUP + p
                for r in range(NREF):
                    li = cv[r][lane] & (HW - 1)
                    start = pl.multiple_of(li & ~(LANES - 1), LANES)
                    cvec = s_v[b, p, pl.ds(start, LANES)]
                    zv = _lane_splat(cvec, li & (LANES - 1))
                    acc[r] = jnp.where(lane_iota == lane, zv, acc[r])
        aoff = pl.multiple_of(gp * LANES, LANES)
        for r in range(NREF):
            att_all[r, pl.ds(aoff, LANES)] = acc[r]
        return 0

    lax.fori_loop(0, GPW // 2, pair, 0)
    pltpu.sync_copy(att_all, attz.at[:, pl.ds(wid * PPW, PPW)])


@functools.cache
def _sc_call():
    return pl.kernel(
        _sc_body,
        out_type=jax.ShapeDtypeStruct((NREF, NPIX), jnp.float32),
        mesh=plsc.VectorSubcoreMesh(core_axis_name="c", subcore_axis_name="s"),
        scratch_types=[
            pltpu.VMEM((NREF, PPW), jnp.int32),
            pltpu.VMEM((2, LANES), jnp.int32),
            pltpu.VMEM((2, GROUP, HW), jnp.float32),
            pltpu.VMEM((NREF, PPW), jnp.float32),
            pltpu.SemaphoreType.DMA,
        ],
    )


def _tc2_body(x_ref, wv_ref, bv_ref, gidx_ref, attz_ref, out_ref):
    vp = lax.dot_general(wv_ref[...].astype(jnp.bfloat16),
                         x_ref[0].astype(jnp.bfloat16),
                         (((1,), (0,)), ((), ())),
                         preferred_element_type=jnp.float32) + bv_ref[...]
    att = 1.0 / (1.0 + jnp.exp(-attz_ref[...] * SCALE))
    lidx_t = lax.transpose(gidx_ref[...] & (HW - 1), (1, 0))
    att_t = lax.transpose(att, (1, 0))
    iota_j = lax.broadcasted_iota(jnp.int32, (1, HW), 1)
    pmat = jnp.zeros((HW, HW), jnp.float32)
    for r in range(NREF):
        pmat = pmat + jnp.where(lidx_t[:, r:r + 1] == iota_j,
                                att_t[:, r:r + 1], 0.0)
    out_ref[0] = lax.dot_general(vp.astype(jnp.bfloat16),
                                 pmat.astype(jnp.bfloat16), NT,
                                 preferred_element_type=jnp.float32)


_tc2_call = pl.pallas_call(
    _tc2_body,
    grid=(B,),
    in_specs=[
        pl.BlockSpec((1, C, HW), lambda i: (i, 0, 0)),
        pl.BlockSpec((C, C), lambda i: (0, 0)),
        pl.BlockSpec((C, 1), lambda i: (0, 0)),
        pl.BlockSpec((NREF, HW), lambda i: (0, i)),
        pl.BlockSpec((NREF, HW), lambda i: (0, i)),
    ],
    out_specs=pl.BlockSpec((1, C, HW), lambda i: (i, 0, 0)),
    out_shape=jax.ShapeDtypeStruct((B, C, HW), jnp.float32),
)


def kernel(x, Wq, bq, Wk, bk, Wv, bv, Wo, bo):
    x3 = x.reshape(B, C, HW)
    s2, gidxt = _tc1_call(x3, Wq, Wk, Wo, bq[None, :], bk[None, :], bo[:, None])
    attz = _sc_call()(s2, gidxt)
    out = _tc2_call(x3, Wv, bv[:, None], gidxt, attz)
    return out.reshape(B, C, H, W)


# speedup vs baseline: 1.5893x; 1.2653x over previous
import functools

import jax
import jax.numpy as jnp
from jax import lax
from jax.experimental import pallas as pl
from jax.experimental.pallas import tpu as pltpu
from jax.experimental.pallas import tpu_sc as plsc

B, C, H, W = 8, 768, 32, 32
HW = H * W
NPIX = B * HW
NREF = 4
LANES = 16
NC, NS = 2, 16
NW = NC * NS
PPW = NPIX // NW
GROUP = 8
GPW = PPW // GROUP
SCALE = 1.0 / float(C) ** 0.5
NT = (((1,), (1,)), ((), ()))


def _tc1_body(x_ref, wq_ref, wk_ref, wo_ref, bq_ref, bk_ref, bo_ref,
              s_ref, gidx_ref):
    b = pl.program_id(0)
    xt = x_ref[0]
    q = lax.dot_general(xt, wq_ref[...], NT,
                        preferred_element_type=jnp.float32) + bq_ref[...]
    k = lax.dot_general(xt.astype(jnp.bfloat16),
                        wk_ref[...].astype(jnp.bfloat16), NT,
                        preferred_element_type=jnp.float32) + bk_ref[...]
    s_ref[...] = lax.dot_general(q.astype(jnp.bfloat16), k.astype(jnp.bfloat16),
                                 NT, preferred_element_type=jnp.float32)
    off = lax.dot_general(wo_ref[...], q, NT,
                          preferred_element_type=jnp.float32) + bo_ref[...]
    p = lax.broadcasted_iota(jnp.int32, (1, HW), 1)
    ypix = (p // W).astype(jnp.float32)
    xpix = (p % W).astype(jnp.float32)
    for r in range(NREF):
        rx = jnp.floor(jnp.clip(xpix + off[2 * r:2 * r + 1, :], 0.0, W - 1.0))
        ry = jnp.floor(jnp.clip(ypix + off[2 * r + 1:2 * r + 2, :], 0.0, H - 1.0))
        gidx_ref[r:r + 1, :] = (b * HW + ry.astype(jnp.int32) * W
                                + rx.astype(jnp.int32))


_tc1_call = pl.pallas_call(
    _tc1_body,
    grid=(B,),
    in_specs=[
        pl.BlockSpec((1, HW, C), lambda i: (i, 0, 0)),
        pl.BlockSpec((C, C), lambda i: (0, 0)),
        pl.BlockSpec((C, C), lambda i: (0, 0)),
        pl.BlockSpec((2 * NREF, C), lambda i: (0, 0)),
        pl.BlockSpec((1, C), lambda i: (0, 0)),
        pl.BlockSpec((1, C), lambda i: (0, 0)),
        pl.BlockSpec((2 * NREF, 1), lambda i: (0, 0)),
    ],
    out_specs=[
        pl.BlockSpec((HW, HW), lambda i: (i, 0)),
        pl.BlockSpec((NREF, HW), lambda i: (0, i)),
    ],
    out_shape=[
        jax.ShapeDtypeStruct((NPIX, HW), jnp.float32),
        jax.ShapeDtypeStruct((NREF, NPIX), jnp.int32),
    ],
)


def _lane_perm(vec, perm):
    return lax.gather(
        vec, perm[:, None],
        lax.GatherDimensionNumbers(offset_dims=(), collapsed_slice_dims=(0,),
                                   start_index_map=(0,)),
        slice_sizes=(1,), mode=lax.GatherScatterMode.PROMISE_IN_BOUNDS)


def _lane_splat(vec, lane):
    return _lane_perm(vec, jnp.broadcast_to(lane, (LANES,)))


def _sc_body(s2, gidxt, attz, idx_all, sidx_v, s_v, att_all, sem_in):
    wid = lax.axis_index("s") * NC + lax.axis_index("c")
    g0 = wid * GPW
    pltpu.sync_copy(gidxt.at[:, pl.ds(wid * PPW, PPW)], idx_all)

    def issue(grp, b):
        base = grp * GROUP
        sidx_v[b, pl.ds(0, LANES)] = (jnp.broadcast_to(base, (LANES,))
                                      + lax.iota(jnp.int32, LANES))
        pltpu.async_copy(s2.at[sidx_v.at[b, pl.ds(0, GROUP)]], s_v.at[b],
                         sem_in)

    def wait_in(b):
        pltpu.make_async_copy(s2.at[sidx_v.at[b, pl.ds(0, GROUP)]],
                              s_v.at[b], sem_in).wait()

    issue(g0, 0)
    lane_iota = lax.iota(jnp.int32, LANES)

    def pair(gp, _):
        acc = [jnp.zeros((LANES,), jnp.float32) for _ in range(NREF)]
        coff = pl.multiple_of(gp * LANES, LANES)
        cv = [idx_all[r, pl.ds(coff, LANES)] for r in range(NREF)]
        for b in range(2):
            g = gp * 2 + b
            grp = g0 + g
            wait_in(b)

            @pl.when(g + 1 < GPW)
            def _():
                issue(grp + 1, 1 - b)

            for p in range(GROUP):
                lane = b * GROUP + p
                for r in range(NREF):
                    li = cv[r][lane] & (HW - 1)
                    start = pl.multiple_of(li & ~(LANES - 1), LANES)
                    cvec = s_v[b, p, pl.ds(start, LANES)]
                    zv = _lane_splat(cvec, li & (LANES - 1))
                    acc[r] = jnp.where(lane_iota == lane, zv, acc[r])
        aoff = pl.multiple_of(gp * LANES, LANES)
        for r in range(NREF):
            att_all[r, pl.ds(aoff, LANES)] = acc[r]
        return 0

    lax.fori_loop(0, GPW // 2, pair, 0)
    pltpu.sync_copy(att_all, attz.at[:, pl.ds(wid * PPW, PPW)])


@functools.cache
def _sc_call():
    return pl.kernel(
        _sc_body,
        out_type=jax.ShapeDtypeStruct((NREF, NPIX), jnp.float32),
        mesh=plsc.VectorSubcoreMesh(core_axis_name="c", subcore_axis_name="s"),
        scratch_types=[
            pltpu.VMEM((NREF, PPW), jnp.int32),
            pltpu.VMEM((2, LANES), jnp.int32),
            pltpu.VMEM((2, GROUP, HW), jnp.float32),
            pltpu.VMEM((NREF, PPW), jnp.float32),
            pltpu.SemaphoreType.DMA,
        ],
    )


def _tc2_body(x_ref, wv_ref, bv_ref, gidx_ref, attz_ref, out_ref):
    vp = lax.dot_general(x_ref[0].astype(jnp.bfloat16),
                         wv_ref[...].astype(jnp.bfloat16), NT,
                         preferred_element_type=jnp.float32) + bv_ref[...]
    att = 1.0 / (1.0 + jnp.exp(-attz_ref[...] * SCALE))
    lidx_t = lax.transpose(gidx_ref[...] & (HW - 1), (1, 0))
    att_t = lax.transpose(att, (1, 0))
    iota_j = lax.broadcasted_iota(jnp.int32, (1, HW), 1)
    pmat = jnp.zeros((HW, HW), jnp.float32)
    for r in range(NREF):
        pmat = pmat + jnp.where(lidx_t[:, r:r + 1] == iota_j,
                                att_t[:, r:r + 1], 0.0)
    out_ref[0] = lax.dot_general(pmat.astype(jnp.bfloat16),
                                 vp.astype(jnp.bfloat16),
                                 (((1,), (0,)), ((), ())),
                                 preferred_element_type=jnp.float32)


_tc2_call = pl.pallas_call(
    _tc2_body,
    grid=(B,),
    in_specs=[
        pl.BlockSpec((1, HW, C), lambda i: (i, 0, 0)),
        pl.BlockSpec((C, C), lambda i: (0, 0)),
        pl.BlockSpec((1, C), lambda i: (0, 0)),
        pl.BlockSpec((NREF, HW), lambda i: (0, i)),
        pl.BlockSpec((NREF, HW), lambda i: (0, i)),
    ],
    out_specs=pl.BlockSpec((1, HW, C), lambda i: (i, 0, 0)),
    out_shape=jax.ShapeDtypeStruct((B, HW, C), jnp.float32),
)


def kernel(x, Wq, bq, Wk, bk, Wv, bv, Wo, bo):
    x_pm = x.transpose(0, 2, 3, 1).reshape(B, HW, C)
    s2, gidxt = _tc1_call(x_pm, Wq, Wk, Wo, bq[None, :], bk[None, :],
                          bo[:, None])
    attz = _sc_call()(s2, gidxt)
    out_pm = _tc2_call(x_pm, Wv, bv[None, :], gidxt, attz)
    return out_pm.reshape(B, H, W, C).transpose(0, 3, 1, 2)
